# baseline (speedup 1.0000x reference)
import functools

import jax
import jax.numpy as jnp
from jax import lax
from jax.experimental import pallas as pl
from jax.experimental.pallas import tpu as pltpu
from jax.experimental.pallas import tpu_sc as plsc


def _make_sc_kernel(B, S, D):
    info = plsc.get_sparse_core_info()
    NC, NS = info.num_cores, info.num_subcores
    NW = NC * NS
    RPW = S // NW
    mesh = plsc.VectorSubcoreMesh(core_axis_name="c", subcore_axis_name="s")

    @functools.partial(
        pl.kernel,
        out_type=jax.ShapeDtypeStruct((B, S, D), jnp.float32),
        mesh=mesh,
        scratch_types=[
            pltpu.SemaphoreType.DMA((B,)),
        ],
    )
    def k(w_hbm, out_hbm, sem):
        wid = lax.axis_index("s") * NC + lax.axis_index("c")
        base = wid * RPW
        copies = [
            pltpu.make_async_copy(
                w_hbm.at[pl.ds(base, RPW), :],
                out_hbm.at[b, pl.ds(base, RPW), :],
                sem.at[b],
            )
            for b in range(B)
        ]
        for cp in copies:
            cp.start()
        for cp in copies:
            cp.wait()

    return k


def kernel(x, W):
    B, S = x.shape
    D = W.shape[1]
    assert S % 2048 == 0
    return _make_sc_kernel(B, S, D)(W[:S])


# speedup vs baseline: 30.9497x; 30.9497x over previous
import functools

import jax
import jax.numpy as jnp
from jax import lax
from jax.experimental import pallas as pl
from jax.experimental.pallas import tpu as pltpu
from jax.experimental.pallas import tpu_sc as plsc


def _make_scs_kernel(B, S, D):
    info = plsc.get_sparse_core_info()
    NC = info.num_cores
    RPC = S // NC
    CHS = 1024
    NCH = RPC // CHS
    mesh = plsc.ScalarSubcoreMesh(axis_name="c", num_cores=NC)

    @functools.partial(
        pl.kernel,
        out_type=jax.ShapeDtypeStruct((B, S, D), jnp.float32),
        mesh=mesh,
        scratch_types=[
            pltpu.VMEM_SHARED((CHS, D), jnp.float32),
            pltpu.SemaphoreType.DMA,
        ],
    )
    def k(w_hbm, out_hbm, buf, sem):
        cid = lax.axis_index("c")
        base = cid * RPC
        for c in range(NCH):
            start = base + c * CHS
            pltpu.sync_copy(w_hbm.at[pl.ds(start, CHS), :], buf)
            for b in range(B):
                pltpu.sync_copy(buf, out_hbm.at[b, pl.ds(start, CHS), :])

    return k


def kernel(x, W):
    B, S = x.shape
    D = W.shape[1]
    assert S % 2048 == 0
    return _make_scs_kernel(B, S, D)(W[:S])


# speedup vs baseline: 54.4054x; 1.7579x over previous
import functools

import jax
import jax.numpy as jnp
from jax import lax
from jax.experimental import pallas as pl
from jax.experimental.pallas import tpu as pltpu
from jax.experimental.pallas import tpu_sc as plsc


def _make_sc_kernel(B, S, D, SCS_ROWS):
    info = plsc.get_sparse_core_info()
    NC, NS = info.num_cores, info.num_subcores
    NW = NC * NS
    TEC_ROWS = S - SCS_ROWS
    RPW = TEC_ROWS // NW
    CH = 32
    NCH = RPW // CH
    assert RPW % CH == 0
    RPC = SCS_ROWS // NC
    CHS = 512
    NCHS = RPC // CHS
    assert RPC % CHS == 0

    scs_mesh = plsc.ScalarSubcoreMesh(axis_name="c", num_cores=NC)
    vec_mesh = plsc.VectorSubcoreMesh(core_axis_name="c", subcore_axis_name="s")

    def vec_body(w_hbm, out_hbm, tbuf, sbuf):
        del sbuf
        wid = lax.axis_index("s") * NC + lax.axis_index("c")
        base = wid * RPW
        for c in range(NCH):
            start = base + c * CH
            pltpu.sync_copy(w_hbm.at[pl.ds(start, CH), :], tbuf)
            for b in range(B):
                pltpu.sync_copy(tbuf, out_hbm.at[b, pl.ds(start, CH), :])

    def scs_body(w_hbm, out_hbm, tbuf, sbuf):
        del tbuf
        cid = lax.axis_index("c")
        base = TEC_ROWS + cid * RPC
        for c in range(NCHS):
            start = base + c * CHS
            pltpu.sync_copy(w_hbm.at[pl.ds(start, CHS), :], sbuf)
            for b in range(B):
                pltpu.sync_copy(sbuf, out_hbm.at[b, pl.ds(start, CHS), :])

    return pl.kernel(
        body=[scs_body, vec_body],
        mesh=[scs_mesh, vec_mesh],
        out_type=jax.ShapeDtypeStruct((B, S, D), jnp.float32),
        scratch_types=[
            pltpu.VMEM((CH, D), jnp.float32) @ vec_mesh,
            pltpu.VMEM_SHARED((CHS, D), jnp.float32),
        ],
    )


def kernel(x, W):
    B, S = x.shape
    D = W.shape[1]
    assert S % 2048 == 0
    return _make_sc_kernel(B, S, D, SCS_ROWS=3072)(W[:S])


# speedup vs baseline: 55.6353x; 1.0226x over previous
import functools

import jax
import jax.numpy as jnp
from jax import lax
from jax.experimental import pallas as pl
from jax.experimental.pallas import tpu as pltpu
from jax.experimental.pallas import tpu_sc as plsc


def _make_sc_kernel(B, S, D):
    info = plsc.get_sparse_core_info()
    NC, NS = info.num_cores, info.num_subcores
    NW = NC * NS
    RPW = S // NW
    CH = 64
    NCH = RPW // CH
    assert RPW % CH == 0
    mesh = plsc.VectorSubcoreMesh(core_axis_name="c", subcore_axis_name="s")

    @functools.partial(
        pl.kernel,
        out_type=jax.ShapeDtypeStruct((B, S, D), jnp.float32),
        mesh=mesh,
        scratch_types=[
            pltpu.VMEM((CH, D), jnp.float32),
        ],
    )
    def k(w_hbm, out_hbm, buf):
        wid = lax.axis_index("s") * NC + lax.axis_index("c")
        base = wid * RPW
        for c in range(NCH):
            start = base + c * CH
            pltpu.sync_copy(w_hbm.at[pl.ds(start, CH), :], buf)
            for b in range(B):
                pltpu.sync_copy(buf, out_hbm.at[b, pl.ds(start, CH), :])

    return k


def kernel(x, W):
    B, S = x.shape
    D = W.shape[1]
    assert S % 2048 == 0
    return _make_sc_kernel(B, S, D)(W[:S])
